# Initial kernel scaffold; baseline (speedup 1.0000x reference)
import jax
import jax.numpy as jnp
from jax.experimental import pallas as pl


def kernel(hidden_states, gate_w, W0, b0, W1, b1, Wo, bo, sW0, sb0, sW1, sb1, sWo, sbo, sg_w):
    raise NotImplementedError("write your pallas kernel here")


# speedup vs baseline: 1.1146x; 1.1146x over previous
import jax
import jax.numpy as jnp
from jax.experimental import pallas as pl


def _router_body(hs_ref, gw_ref, logits_ref, dw_ref):
    hs = hs_ref[...]
    l9 = jnp.dot(hs, gw_ref[...], preferred_element_type=jnp.float32)
    T, EP = l9.shape
    E = EP - 1
    logits_ref[...] = l9[:, :E]
    lane = jax.lax.broadcasted_iota(jnp.int32, (T, EP), 1)
    moe = lane < E
    lm = jnp.where(moe, l9, -1e30)
    mx = jnp.max(lm, axis=1, keepdims=True)
    ex = jnp.where(moe, jnp.exp(lm - mx), 0.0)
    rw = ex / jnp.sum(ex, axis=1, keepdims=True)
    m1 = jnp.max(rw, axis=1, keepdims=True)
    e0 = jnp.min(jnp.where(rw == m1, lane, EP), axis=1, keepdims=True)
    rw2 = jnp.where(lane == e0, -1.0, rw)
    m2 = jnp.max(rw2, axis=1, keepdims=True)
    e1 = jnp.min(jnp.where(rw2 == m2, lane, EP), axis=1, keepdims=True)
    den = m1 + m2
    w0 = m1 / den
    w1 = m2 / den
    sgate = jax.nn.sigmoid(l9[:, E:EP])
    dw_ref[...] = (jnp.where(lane == e0, w0, 0.0)
                   + jnp.where(lane == e1, w1, 0.0)
                   + jnp.where(lane == E, sgate, 0.0))


def _ffn_body(hs_ref, w0_ref, w1_ref, wo_ref, b0_ref, b1_ref, bo_ref, dw_ref,
              out_ref):
    e = pl.program_id(1)
    x = hs_ref[...].astype(jnp.bfloat16)
    h0 = jnp.dot(x, w0_ref[0], preferred_element_type=jnp.float32) + b0_ref[0]
    h1 = jnp.dot(x, w1_ref[0], preferred_element_type=jnp.float32) + b1_ref[0]
    inter = (h0 * jax.nn.sigmoid(h0) * h1).astype(jnp.bfloat16)
    out = jnp.dot(inter, wo_ref[0], preferred_element_type=jnp.float32) + bo_ref[0]
    lane = jax.lax.broadcasted_iota(jnp.int32, dw_ref.shape, 1)
    wcol = jnp.sum(jnp.where(lane == e, dw_ref[...], 0.0), axis=1, keepdims=True)
    contrib = out * wcol

    @pl.when(e == 0)
    def _():
        out_ref[...] = contrib

    @pl.when(e > 0)
    def _():
        out_ref[...] += contrib


def kernel(hidden_states, gate_w, W0, b0, W1, b1, Wo, bo, sW0, sb0, sW1, sb1,
           sWo, sbo, sg_w):
    b_, s_, h_ = hidden_states.shape
    T = b_ * s_
    E = gate_w.shape[1]
    I = W0.shape[2]
    EP = E + 1
    hs2 = hidden_states.reshape(T, h_)
    gwcat = jnp.concatenate([gate_w, sg_w], axis=1)

    logits, dw = pl.pallas_call(
        _router_body,
        out_shape=[
            jax.ShapeDtypeStruct((T, E), jnp.float32),
            jax.ShapeDtypeStruct((T, EP), jnp.float32),
        ],
    )(hs2, gwcat)

    bf = jnp.bfloat16
    W0c = jnp.concatenate([W0, sW0[None]], axis=0).astype(bf)
    W1c = jnp.concatenate([W1, sW1[None]], axis=0).astype(bf)
    Woc = jnp.concatenate([Wo, sWo[None]], axis=0).astype(bf)
    b0c = jnp.concatenate([b0, sb0[None]], axis=0).reshape(EP, 1, I)
    b1c = jnp.concatenate([b1, sb1[None]], axis=0).reshape(EP, 1, I)
    boc = jnp.concatenate([bo, sbo[None]], axis=0).reshape(EP, 1, h_)

    Bm = 512
    grid = (T // Bm, EP)
    final = pl.pallas_call(
        _ffn_body,
        grid=grid,
        in_specs=[
            pl.BlockSpec((Bm, h_), lambda i, e: (i, 0)),
            pl.BlockSpec((1, h_, I), lambda i, e: (e, 0, 0)),
            pl.BlockSpec((1, h_, I), lambda i, e: (e, 0, 0)),
            pl.BlockSpec((1, I, h_), lambda i, e: (e, 0, 0)),
            pl.BlockSpec((1, 1, I), lambda i, e: (e, 0, 0)),
            pl.BlockSpec((1, 1, I), lambda i, e: (e, 0, 0)),
            pl.BlockSpec((1, 1, h_), lambda i, e: (e, 0, 0)),
            pl.BlockSpec((Bm, EP), lambda i, e: (i, 0)),
        ],
        out_specs=pl.BlockSpec((Bm, h_), lambda i, e: (i, 0)),
        out_shape=jax.ShapeDtypeStruct((T, h_), jnp.float32),
    )(hs2, W0c, W1c, Woc, b0c, b1c, boc, dw)

    return final.reshape(b_, s_, h_), logits
